# Initial kernel scaffold; baseline (speedup 1.0000x reference)
import jax
import jax.numpy as jnp
from jax.experimental import pallas as pl


def kernel(nodes, neigh_idx, features, weight):
    raise NotImplementedError("write your pallas kernel here")


# speedup vs baseline: 3.3356x; 3.3356x over previous
import functools

import jax
import jax.numpy as jnp
from jax import lax
from jax.experimental import pallas as pl
from jax.experimental.pallas import tpu as pltpu
from jax.experimental.pallas import tpu_sc as plsc

B = 16384
D = 128
S = 10
R = S + 1
NC, NS = 2, 16
NW = NC * NS
C = 64
RPW = B // NW
KCH = RPW // C
NCHUNK = B // C

_sc_mesh = plsc.VectorSubcoreMesh(core_axis_name="c", subcore_axis_name="s")


@functools.partial(
    pl.kernel,
    out_type=(
        jax.ShapeDtypeStruct((B, D), jnp.float32),
        jax.ShapeDtypeStruct((B, D), jnp.float32),
    ),
    mesh=_sc_mesh,
    scratch_types=[
        pltpu.VMEM((R, C), jnp.int32),
        pltpu.VMEM((R * C, D), jnp.float32),
        pltpu.VMEM((C, D), jnp.float32),
        pltpu.SemaphoreType.DMA,
    ],
)
def _sc_gather_sum(idx_hbm, feat_hbm, self_out, nsum_out, idx_v, buf_v, acc_v, sem):
    wid = lax.axis_index("s") * NC + lax.axis_index("c")

    def chunk_body(k, carry):
        g = wid * KCH + k
        base = g * C
        pltpu.sync_copy(idx_hbm.at[g], idx_v)
        copies = []
        for j in range(R):
            copies.append(
                pltpu.async_copy(
                    feat_hbm.at[idx_v.at[j]], buf_v.at[pl.ds(j * C, C)], sem
                )
            )
        for cp in copies:
            cp.wait()
        pltpu.sync_copy(buf_v.at[pl.ds(0, C)], self_out.at[pl.ds(base, C)])

        def row_body(r, carry2):
            for l in range(D // 16):
                sl = pl.ds(l * 16, 16)
                v = buf_v[C + r, sl]
                for j in range(2, R):
                    v = v + buf_v[j * C + r, sl]
                acc_v[r, sl] = v
            return carry2

        lax.fori_loop(0, C, row_body, 0, unroll=False)
        pltpu.sync_copy(acc_v, nsum_out.at[pl.ds(base, C)])
        return carry

    lax.fori_loop(0, KCH, chunk_body, 0, unroll=False)


def _tc_body(self_ref, nsum_ref, ws_ref, wn_ref, out_ref):
    z = lax.dot_general(
        ws_ref[...], self_ref[...], (((1,), (1,)), ((), ())),
        preferred_element_type=jnp.float32,
    )
    z += lax.dot_general(
        wn_ref[...], nsum_ref[...], (((1,), (1,)), ((), ())),
        preferred_element_type=jnp.float32,
    )
    out_ref[...] = jnp.maximum(z, 0.0)


_BT = 2048


@jax.jit
def kernel(nodes, neigh_idx, features, weight):
    nodes = nodes.astype(jnp.int32)
    neigh_idx = neigh_idx.astype(jnp.int32)
    idx_all = jnp.concatenate([nodes[:, None], neigh_idx], axis=1)
    idx_chunks = idx_all.reshape(NCHUNK, C, R).transpose(0, 2, 1)

    self_feats, nsum = _sc_gather_sum(idx_chunks, features)

    w_self = weight[:, :D]
    w_neigh = weight[:, D:] * (1.0 / S)

    out = pl.pallas_call(
        _tc_body,
        grid=(B // _BT,),
        in_specs=[
            pl.BlockSpec((_BT, D), lambda i: (i, 0)),
            pl.BlockSpec((_BT, D), lambda i: (i, 0)),
            pl.BlockSpec((D, D), lambda i: (0, 0)),
            pl.BlockSpec((D, D), lambda i: (0, 0)),
        ],
        out_specs=pl.BlockSpec((D, _BT), lambda i: (0, i)),
        out_shape=jax.ShapeDtypeStruct((D, B), jnp.float32),
    )(self_feats, nsum, w_self, w_neigh)
    return out


# speedup vs baseline: 3.8895x; 1.1661x over previous
import functools

import jax
import jax.numpy as jnp
from jax import lax
from jax.experimental import pallas as pl
from jax.experimental.pallas import tpu as pltpu
from jax.experimental.pallas import tpu_sc as plsc

B = 16384
D = 128
S = 10
R = S + 1
NC, NS = 2, 16
NW = NC * NS
C = 32
RPW = B // NW
KCH = RPW // C
NCHUNK = B // C
LANES = 16

_sc_mesh = plsc.VectorSubcoreMesh(core_axis_name="c", subcore_axis_name="s")


@functools.partial(
    pl.kernel,
    out_type=(
        jax.ShapeDtypeStruct((B, D), jnp.float32),
        jax.ShapeDtypeStruct((B, D), jnp.float32),
    ),
    mesh=_sc_mesh,
    scratch_types=[
        pltpu.VMEM((R, C), jnp.int32),
        pltpu.VMEM((R, C), jnp.int32),
        pltpu.VMEM((R * C, D), jnp.float32),
        pltpu.VMEM((R * C, D), jnp.float32),
        pltpu.VMEM((C, D), jnp.float32),
        pltpu.VMEM((C, D), jnp.float32),
        pltpu.SemaphoreType.DMA,
        pltpu.SemaphoreType.DMA,
        pltpu.SemaphoreType.DMA,
        pltpu.SemaphoreType.DMA,
        pltpu.SemaphoreType.DMA,
        pltpu.SemaphoreType.DMA,
    ],
)
def _sc_gather_sum(idx_hbm, feat_hbm, self_out, nsum_out,
                   idx0, idx1, buf0, buf1, acc0, acc1,
                   g0, g1, s0, s1, a0, a1):
    wid = lax.axis_index("s") * NC + lax.axis_index("c")
    idx = [idx0, idx1]
    buf = [buf0, buf1]
    acc = [acc0, acc1]
    gsem = [g0, g1]
    ssem = [s0, s1]
    asem = [a0, a1]

    def issue_chunk(k, b):
        g = wid * KCH + k
        pltpu.sync_copy(idx_hbm.at[g], idx[b])
        return [
            pltpu.async_copy(
                feat_hbm.at[idx[b].at[j]], buf[b].at[pl.ds(j * C, C)], gsem[b]
            )
            for j in range(R)
        ]

    def accumulate(b):
        src = buf[b]
        dst = acc[b]

        def row_body(r, carry):
            for l in range(D // LANES):
                sl = pl.ds(l * LANES, LANES)
                v = src[C + r, sl]
                for j in range(2, R):
                    v = v + src[j * C + r, sl]
                dst[r, sl] = v
            return carry

        lax.fori_loop(0, C, row_body, 0, unroll=False)

    pend_gather = [None, None]
    pend_out = [None, None]

    pend_gather[0] = issue_chunk(0, 0)
    for k in range(KCH):
        b = k % 2
        nb = 1 - b
        if k + 1 < KCH:
            if pend_out[nb] is not None:
                for cp in pend_out[nb]:
                    cp.wait()
                pend_out[nb] = None
            pend_gather[nb] = issue_chunk(k + 1, nb)
        for cp in pend_gather[b]:
            cp.wait()
        base = (wid * KCH + k) * C
        sd = pltpu.async_copy(
            buf[b].at[pl.ds(0, C)], self_out.at[pl.ds(base, C)], ssem[b]
        )
        accumulate(b)
        ad = pltpu.async_copy(acc[b], nsum_out.at[pl.ds(base, C)], asem[b])
        pend_out[b] = (sd, ad)

    for b in range(2):
        if pend_out[b] is not None:
            for cp in pend_out[b]:
                cp.wait()


def _tc_body(self_ref, nsum_ref, ws_ref, wn_ref, out_ref):
    z = lax.dot_general(
        ws_ref[...], self_ref[...], (((1,), (1,)), ((), ())),
        preferred_element_type=jnp.float32,
    )
    z += lax.dot_general(
        wn_ref[...], nsum_ref[...], (((1,), (1,)), ((), ())),
        preferred_element_type=jnp.float32,
    )
    out_ref[...] = jnp.maximum(z, 0.0)


_BT = 2048


@jax.jit
def kernel(nodes, neigh_idx, features, weight):
    nodes = nodes.astype(jnp.int32)
    neigh_idx = neigh_idx.astype(jnp.int32)
    idx_all = jnp.concatenate([nodes[:, None], neigh_idx], axis=1)
    idx_chunks = idx_all.reshape(NCHUNK, C, R).transpose(0, 2, 1)

    self_feats, nsum = _sc_gather_sum(idx_chunks, features)

    w_self = weight[:, :D]
    w_neigh = weight[:, D:] * (1.0 / S)

    out = pl.pallas_call(
        _tc_body,
        grid=(B // _BT,),
        in_specs=[
            pl.BlockSpec((_BT, D), lambda i: (i, 0)),
            pl.BlockSpec((_BT, D), lambda i: (i, 0)),
            pl.BlockSpec((D, D), lambda i: (0, 0)),
            pl.BlockSpec((D, D), lambda i: (0, 0)),
        ],
        out_specs=pl.BlockSpec((D, _BT), lambda i: (0, i)),
        out_shape=jax.ShapeDtypeStruct((D, B), jnp.float32),
    )(self_feats, nsum, w_self, w_neigh)
    return out
